# baseline (speedup 1.0000x reference)
import math

import jax
import jax.numpy as jnp
from jax import lax
from jax.experimental import pallas as pl
from jax.experimental.pallas import tpu as pltpu
from jax.experimental.pallas import tpu_sc as plsc

VOCAB = 100000
D = 128
SCALE = math.sqrt(float(D))

NC = 2
NS = 16
NW = NC * NS

B_TOTAL = 4096 * 200
B_PER_W = B_TOTAL // NW
CHUNK = 128
N_CHUNKS = B_PER_W // CHUNK
NBUF = 5
N_SUPER = N_CHUNKS // NBUF


def _scale_kernel(t_ref, o_ref):
    o_ref[...] = t_ref[...] * SCALE


def _scaled_table(table):
    tb = 1000
    return pl.pallas_call(
        _scale_kernel,
        out_shape=jax.ShapeDtypeStruct((VOCAB, D), jnp.float32),
        grid=(VOCAB // tb,),
        in_specs=[pl.BlockSpec((tb, D), lambda i: (i, 0))],
        out_specs=pl.BlockSpec((tb, D), lambda i: (i, 0)),
    )(table)


_mesh = plsc.VectorSubcoreMesh(core_axis_name="c", subcore_axis_name="s",
                               num_cores=NC, num_subcores=NS)


@jax.jit
def _gather(table, idx):
    @pl.kernel(
        out_type=jax.ShapeDtypeStruct((B_TOTAL, D), jnp.float32),
        mesh=_mesh,
        scratch_types=[
            pltpu.VMEM((B_PER_W,), jnp.int32),
            pltpu.VMEM((NBUF, CHUNK, D), jnp.float32),
            [pltpu.SemaphoreType.DMA] * NBUF,
            [pltpu.SemaphoreType.DMA] * NBUF,
        ],
    )
    def k(table_hbm, idx_hbm, out_hbm, idx_v, rows_v, sem_in, sem_out):
        wid = lax.axis_index("s") * NC + lax.axis_index("c")
        base = wid * B_PER_W
        pltpu.sync_copy(idx_hbm.at[pl.ds(base, B_PER_W)], idx_v)

        def gather_start(g, b):
            pltpu.async_copy(
                table_hbm.at[idx_v.at[pl.ds(g * CHUNK, CHUNK)]],
                rows_v.at[b], sem_in[b],
            )

        def out_start(g, b):
            pltpu.async_copy(
                rows_v.at[b], out_hbm.at[pl.ds(base + g * CHUNK, CHUNK)],
                sem_out[b],
            )

        def wait(sem, b):
            pltpu.make_async_copy(
                table_hbm.at[pl.ds(0, CHUNK)], rows_v.at[b], sem[b]
            ).wait()

        for b in range(NBUF):
            gather_start(b, b)

        def body(t, carry):
            g0 = t * NBUF
            for b in range(NBUF):
                wait(sem_in, b)
                out_start(g0 + b, b)
            for b in range(NBUF):
                wait(sem_out, b)
                gather_start(g0 + NBUF + b, b)
            return carry

        lax.fori_loop(0, N_SUPER - 1, body, 0)

        g0 = (N_SUPER - 1) * NBUF
        for b in range(NBUF):
            wait(sem_in, b)
            out_start(g0 + b, b)
        for b in range(NBUF):
            wait(sem_out, b)

    return k(table, idx)


def kernel(tokens, table):
    idx = tokens.reshape(-1).astype(jnp.int32)
    out = _gather(_scaled_table(table), idx)
    return out.reshape(tokens.shape[0], tokens.shape[1], D)


# speedup vs baseline: 1.2244x; 1.2244x over previous
import math

import jax
import jax.numpy as jnp
from jax import lax
from jax.experimental import pallas as pl
from jax.experimental.pallas import tpu as pltpu
from jax.experimental.pallas import tpu_sc as plsc

VOCAB = 100000
D = 128
SCALE = math.sqrt(float(D))

NC = 2
NS = 16
NW = NC * NS

B_TOTAL = 4096 * 200
B_PER_W = B_TOTAL // NW
CHUNK = 128
N_CHUNKS = B_PER_W // CHUNK
NBUF = 5
N_SUPER = N_CHUNKS // NBUF


def _scale_kernel(t_ref, o_ref):
    o_ref[...] = t_ref[...] * SCALE


def _scaled_table(table):
    tb = 1000
    return pl.pallas_call(
        _scale_kernel,
        out_shape=jax.ShapeDtypeStruct((VOCAB, D), jnp.float32),
        grid=(VOCAB // tb,),
        in_specs=[pl.BlockSpec((tb, D), lambda i: (i, 0))],
        out_specs=pl.BlockSpec((tb, D), lambda i: (i, 0)),
    )(table)


_mesh = plsc.VectorSubcoreMesh(core_axis_name="c", subcore_axis_name="s",
                               num_cores=NC, num_subcores=NS)


@jax.jit
def _gather(table, idx):
    @pl.kernel(
        out_type=jax.ShapeDtypeStruct((B_TOTAL, D), jnp.float32),
        mesh=_mesh,
        scratch_types=[
            pltpu.VMEM((B_PER_W,), jnp.int32),
            pltpu.VMEM((NBUF, CHUNK, D), jnp.float32),
            [pltpu.SemaphoreType.DMA] * NBUF,
            [pltpu.SemaphoreType.DMA] * NBUF,
        ],
    )
    def k(table_hbm, idx_hbm, out_hbm, idx_v, rows_v, sem_in, sem_out):
        wid = lax.axis_index("s") * NC + lax.axis_index("c")
        base = wid * B_PER_W
        pltpu.sync_copy(idx_hbm.at[pl.ds(base, B_PER_W)], idx_v)

        def gather_start(g, b):
            pltpu.async_copy(
                table_hbm.at[idx_v.at[pl.ds(g * CHUNK, CHUNK)]],
                rows_v.at[b], sem_in[b],
            )

        def out_start(g, b):
            pltpu.async_copy(
                rows_v.at[b], out_hbm.at[pl.ds(base + g * CHUNK, CHUNK)],
                sem_out[b],
            )

        def wait(sem, b):
            pltpu.make_async_copy(
                table_hbm.at[pl.ds(0, CHUNK)], rows_v.at[b], sem[b]
            ).wait()

        for b in range(NBUF):
            gather_start(b, b)

        def scale_buf(b):
            def row_body(r, carry):
                for j in range(D // 16):
                    sl = (b, r, pl.ds(j * 16, 16))
                    rows_v[sl] = rows_v[sl] * SCALE
                return carry
            lax.fori_loop(0, CHUNK, row_body, 0)

        def body(t, carry):
            g0 = t * NBUF
            for b in range(NBUF):
                wait(sem_in, b)
                scale_buf(b)
                out_start(g0 + b, b)
            for b in range(NBUF):
                wait(sem_out, b)
                gather_start(g0 + NBUF + b, b)
            return carry

        lax.fori_loop(0, N_SUPER - 1, body, 0)

        g0 = (N_SUPER - 1) * NBUF
        for b in range(NBUF):
            wait(sem_in, b)
            scale_buf(b)
            out_start(g0 + b, b)
        for b in range(NBUF):
            wait(sem_out, b)

    return k(table, idx)


def kernel(tokens, table):
    idx = tokens.reshape(-1).astype(jnp.int32)
    out = _gather(table, idx)
    return out.reshape(tokens.shape[0], tokens.shape[1], D)


# speedup vs baseline: 1.2303x; 1.0048x over previous
import math

import jax
import jax.numpy as jnp
from jax import lax
from jax.experimental import pallas as pl
from jax.experimental.pallas import tpu as pltpu
from jax.experimental.pallas import tpu_sc as plsc

VOCAB = 100000
D = 128
SCALE = math.sqrt(float(D))

NC = 2
NS = 16
NW = NC * NS

B_TOTAL = 4096 * 200
B_PER_W = B_TOTAL // NW
CHUNK = 256
N_CHUNKS = B_PER_W // CHUNK
NBUF = 2
N_SUPER = N_CHUNKS // NBUF


def _scale_kernel(t_ref, o_ref):
    o_ref[...] = t_ref[...] * SCALE


def _scaled_table(table):
    tb = 1000
    return pl.pallas_call(
        _scale_kernel,
        out_shape=jax.ShapeDtypeStruct((VOCAB, D), jnp.float32),
        grid=(VOCAB // tb,),
        in_specs=[pl.BlockSpec((tb, D), lambda i: (i, 0))],
        out_specs=pl.BlockSpec((tb, D), lambda i: (i, 0)),
    )(table)


_mesh = plsc.VectorSubcoreMesh(core_axis_name="c", subcore_axis_name="s",
                               num_cores=NC, num_subcores=NS)


@jax.jit
def _gather(table, idx):
    @pl.kernel(
        out_type=jax.ShapeDtypeStruct((B_TOTAL, D), jnp.float32),
        mesh=_mesh,
        scratch_types=[
            pltpu.VMEM((B_PER_W,), jnp.int32),
            pltpu.VMEM((NBUF, CHUNK, D), jnp.float32),
            [pltpu.SemaphoreType.DMA] * NBUF,
            [pltpu.SemaphoreType.DMA] * NBUF,
        ],
    )
    def k(table_hbm, idx_hbm, out_hbm, idx_v, rows_v, sem_in, sem_out):
        wid = lax.axis_index("s") * NC + lax.axis_index("c")
        base = wid * B_PER_W
        pltpu.sync_copy(idx_hbm.at[pl.ds(base, B_PER_W)], idx_v)

        def gather_start(g, b):
            pltpu.async_copy(
                table_hbm.at[idx_v.at[pl.ds(g * CHUNK, CHUNK)]],
                rows_v.at[b], sem_in[b],
            )

        def out_start(g, b):
            pltpu.async_copy(
                rows_v.at[b], out_hbm.at[pl.ds(base + g * CHUNK, CHUNK)],
                sem_out[b],
            )

        def wait(sem, b):
            pltpu.make_async_copy(
                table_hbm.at[pl.ds(0, CHUNK)], rows_v.at[b], sem[b]
            ).wait()

        for b in range(NBUF):
            gather_start(b, b)

        def scale_buf(b):
            def row_body(r, carry):
                for j in range(D // 16):
                    sl = (b, r, pl.ds(j * 16, 16))
                    rows_v[sl] = rows_v[sl] * SCALE
                return carry
            lax.fori_loop(0, CHUNK, row_body, 0)

        def body(t, carry):
            g0 = t * NBUF
            for b in range(NBUF):
                wait(sem_in, b)
                scale_buf(b)
                out_start(g0 + b, b)
            for b in range(NBUF):
                wait(sem_out, b)
                gather_start(g0 + NBUF + b, b)
            return carry

        lax.fori_loop(0, N_SUPER - 1, body, 0)

        g0 = (N_SUPER - 1) * NBUF
        for b in range(NBUF):
            wait(sem_in, b)
            scale_buf(b)
            out_start(g0 + b, b)
        for b in range(NBUF):
            wait(sem_out, b)

    return k(table, idx)


def kernel(tokens, table):
    idx = tokens.reshape(-1).astype(jnp.int32)
    out = _gather(table, idx)
    return out.reshape(tokens.shape[0], tokens.shape[1], D)
